# baseline (speedup 1.0000x reference)
import functools

import jax
import jax.numpy as jnp
from jax import lax
from jax.experimental import pallas as pl
from jax.experimental.pallas import tpu as pltpu
from jax.experimental.pallas import tpu_sc as plsc

MAX_LEN = 200
EMB_DIM = 128
BATCH = 4096
NTOK = BATCH * MAX_LEN
HIDDEN = 32

_info = plsc.get_sparse_core_info()
_NC, _NS = _info.num_cores, _info.num_subcores
NW = _NC * _NS
ROWS_PER_W = NTOK // NW
CH = 128
NCHUNK = ROWS_PER_W // CH


def _make_sc_gather():
    mesh = plsc.VectorSubcoreMesh(core_axis_name="c", subcore_axis_name="s")

    @functools.partial(
        pl.kernel,
        mesh=mesh,
        out_type=jax.ShapeDtypeStruct((NTOK, EMB_DIM), jnp.float32),
        scratch_types=[
            pltpu.VMEM((ROWS_PER_W,), jnp.int32),
            pltpu.VMEM((CH, EMB_DIM), jnp.float32),
            pltpu.SemaphoreType.DMA,
        ],
    )
    def gather_k(idx_hbm, table_hbm, out_hbm, idx_v, rows_v, sem):
        wid = lax.axis_index("s") * _NC + lax.axis_index("c")
        base = wid * ROWS_PER_W
        pltpu.sync_copy(idx_hbm.at[pl.ds(base, ROWS_PER_W)], idx_v)

        def body(c, carry):
            off = c * CH
            pltpu.async_copy(
                table_hbm.at[idx_v.at[pl.ds(off, CH)]], rows_v, sem
            ).wait()
            pltpu.sync_copy(rows_v, out_hbm.at[pl.ds(base + off, CH)])
            return carry

        lax.fori_loop(0, NCHUNK, body, 0)

    return gather_k


_sc_gather = _make_sc_gather()

BB = 512


def _mlp_body(s_ref, w1_ref, b1_ref, w2_ref, b2_ref, out_ref, acc_ref):
    t = pl.program_id(1)
    partial = jnp.dot(s_ref[0], w1_ref[t], preferred_element_type=jnp.float32)

    @pl.when(t == 0)
    def _():
        acc_ref[...] = jnp.zeros_like(acc_ref)

    acc_ref[...] += partial

    @pl.when(t == MAX_LEN - 1)
    def _():
        h = jnp.maximum(acc_ref[...] + b1_ref[...], 0.0)
        o = jnp.sum(h * w2_ref[...], axis=1, keepdims=True) + b2_ref[...]
        out_ref[...] = jax.nn.sigmoid(o)


def _mlp(staged3, W1r, b1r, W2r, b2r):
    return pl.pallas_call(
        _mlp_body,
        grid=(BATCH // BB, MAX_LEN),
        in_specs=[
            pl.BlockSpec((1, BB, EMB_DIM), lambda i, t: (t, i, 0)),
            pl.BlockSpec((MAX_LEN, EMB_DIM, HIDDEN), lambda i, t: (0, 0, 0)),
            pl.BlockSpec((1, HIDDEN), lambda i, t: (0, 0)),
            pl.BlockSpec((1, HIDDEN), lambda i, t: (0, 0)),
            pl.BlockSpec((1, 1), lambda i, t: (0, 0)),
        ],
        out_specs=pl.BlockSpec((BB, 1), lambda i, t: (i, 0)),
        out_shape=jax.ShapeDtypeStruct((BATCH, 1), jnp.float32),
        scratch_shapes=[pltpu.VMEM((BB, HIDDEN), jnp.float32)],
    )(staged3, W1r, b1r, W2r, b2r)


def kernel(x, emb, W1, b1, W2, b2):
    idx = x.astype(jnp.int32).T.reshape(-1)
    staged = _sc_gather(idx, emb)
    staged3 = staged.reshape(MAX_LEN, BATCH, EMB_DIM)
    W1r = W1.reshape(MAX_LEN, EMB_DIM, HIDDEN)
    return _mlp(
        staged3,
        W1r,
        b1.reshape(1, HIDDEN),
        W2.reshape(1, HIDDEN),
        b2.reshape(1, 1),
    )


# speedup vs baseline: 2.3087x; 2.3087x over previous
import functools

import jax
import jax.numpy as jnp
from jax import lax
from jax.experimental import pallas as pl
from jax.experimental.pallas import tpu as pltpu
from jax.experimental.pallas import tpu_sc as plsc

MAX_LEN = 200
EMB_DIM = 128
BATCH = 4096
NTOK = BATCH * MAX_LEN
HIDDEN = 32

_info = plsc.get_sparse_core_info()
_NC, _NS = _info.num_cores, _info.num_subcores
NW = _NC * _NS
ROWS_PER_W = NTOK // NW
CH = 128
NCHUNK = ROWS_PER_W // CH


def _make_sc_gather():
    mesh = plsc.VectorSubcoreMesh(core_axis_name="c", subcore_axis_name="s")

    @functools.partial(
        pl.kernel,
        mesh=mesh,
        out_type=jax.ShapeDtypeStruct((NTOK, EMB_DIM), jnp.float32),
        scratch_types=[
            pltpu.VMEM((ROWS_PER_W,), jnp.int32),
            pltpu.VMEM((CH, EMB_DIM), jnp.float32),
            pltpu.SemaphoreType.DMA,
        ],
    )
    def gather_k(idx_hbm, table_hbm, out_hbm, idx_v, rows_v, sem):
        wid = lax.axis_index("s") * _NC + lax.axis_index("c")
        base = wid * ROWS_PER_W
        pltpu.sync_copy(idx_hbm.at[pl.ds(base, ROWS_PER_W)], idx_v)

        def body(c, carry):
            off = c * CH
            pltpu.async_copy(
                table_hbm.at[idx_v.at[pl.ds(off, CH)]], rows_v, sem
            ).wait()
            pltpu.sync_copy(rows_v, out_hbm.at[pl.ds(base + off, CH)])
            return carry

        lax.fori_loop(0, NCHUNK, body, 0)

    return gather_k


_sc_gather = _make_sc_gather()

BB = 512
TT = 25
NT = MAX_LEN // TT


def _mlp_body(s_ref, w1_ref, b1_ref, w2_ref, b2_ref, out_ref, acc_ref):
    tc = pl.program_id(1)
    partial = jnp.dot(s_ref[0], w1_ref[0], preferred_element_type=jnp.float32)
    for tt in range(1, TT):
        partial += jnp.dot(
            s_ref[tt], w1_ref[tt], preferred_element_type=jnp.float32
        )

    @pl.when(tc == 0)
    def _():
        acc_ref[...] = jnp.zeros_like(acc_ref)

    acc_ref[...] += partial

    @pl.when(tc == NT - 1)
    def _():
        h = jnp.maximum(acc_ref[...] + b1_ref[...], 0.0)
        o = jnp.sum(h * w2_ref[...], axis=1, keepdims=True) + b2_ref[...]
        out_ref[...] = jax.nn.sigmoid(o)


def _mlp(staged3, W1r, b1r, W2r, b2r):
    return pl.pallas_call(
        _mlp_body,
        grid=(BATCH // BB, NT),
        in_specs=[
            pl.BlockSpec((TT, BB, EMB_DIM), lambda i, t: (t, i, 0)),
            pl.BlockSpec((TT, EMB_DIM, HIDDEN), lambda i, t: (t, 0, 0)),
            pl.BlockSpec((1, HIDDEN), lambda i, t: (0, 0)),
            pl.BlockSpec((1, HIDDEN), lambda i, t: (0, 0)),
            pl.BlockSpec((1, 1), lambda i, t: (0, 0)),
        ],
        out_specs=pl.BlockSpec((BB, 1), lambda i, t: (i, 0)),
        out_shape=jax.ShapeDtypeStruct((BATCH, 1), jnp.float32),
        scratch_shapes=[pltpu.VMEM((BB, HIDDEN), jnp.float32)],
    )(staged3, W1r, b1r, W2r, b2r)


def kernel(x, emb, W1, b1, W2, b2):
    idx = x.astype(jnp.int32).T.reshape(-1)
    staged = _sc_gather(idx, emb)
    staged3 = staged.reshape(MAX_LEN, BATCH, EMB_DIM)
    W1r = W1.reshape(MAX_LEN, EMB_DIM, HIDDEN)
    return _mlp(
        staged3,
        W1r,
        b1.reshape(1, HIDDEN),
        W2.reshape(1, HIDDEN),
        b2.reshape(1, 1),
    )
